# baseline (speedup 1.0000x reference)
import functools

import jax
import jax.numpy as jnp
from jax import lax
from jax.experimental import pallas as pl
from jax.experimental.pallas import tpu as pltpu
from jax.experimental.pallas import tpu_sc as plsc

N = 10000
E = 320000
D = 128
H = 128

_NC = 2
_NS = 16
_NW = _NC * _NS
_CK = 128
_NCHUNK = E // _CK
_NJ = (_NCHUNK + _NW - 1) // _NW


def _silu(x):
    return x * jax.nn.sigmoid(x)


def _proj_body(h_ref, w1_ref, b1_ref, ha_ref, hb_ref):
    h = h_ref[...]
    w1a = w1_ref[0:D, :]
    w1b = w1_ref[D:2 * D, :]
    ha = jnp.dot(h, w1a, preferred_element_type=jnp.float32) + b1_ref[...]
    hb = jnp.dot(h, w1b, preferred_element_type=jnp.float32)
    ha_ref[...] = ha.astype(jnp.bfloat16)
    hb_ref[...] = hb.astype(jnp.bfloat16)


def _proj(h, W1, b1):
    BN = 2000
    return pl.pallas_call(
        _proj_body,
        grid=(N // BN,),
        in_specs=[
            pl.BlockSpec((BN, D), lambda i: (i, 0)),
            pl.BlockSpec((2 * D + 1, H), lambda i: (0, 0)),
            pl.BlockSpec((1, H), lambda i: (0, 0)),
        ],
        out_specs=[
            pl.BlockSpec((BN, H), lambda i: (i, 0)),
            pl.BlockSpec((BN, H), lambda i: (i, 0)),
        ],
        out_shape=[
            jax.ShapeDtypeStruct((N, H), jnp.bfloat16),
            jax.ShapeDtypeStruct((N, H), jnp.bfloat16),
        ],
    )(h, W1, b1)


def _gather_body(ha_hbm, hb_hbm, row2d_hbm, col2d_hbm, ga_hbm, gb_hbm,
                 rowv, colv, bufa, bufb, sema, semb, semo):
    wid = lax.axis_index("s") * _NC + lax.axis_index("c")

    def chunk_step(j, carry):
        chunk = wid + _NW * j

        @pl.when(chunk < _NCHUNK)
        def _():
            pltpu.sync_copy(row2d_hbm.at[pl.ds(chunk, 1)], rowv)
            pltpu.sync_copy(col2d_hbm.at[pl.ds(chunk, 1)], colv)
            cpa = pltpu.async_copy(ha_hbm.at[rowv.at[0]], bufa, sema)
            cpb = pltpu.async_copy(hb_hbm.at[colv.at[0]], bufb, semb)
            cpa.wait()
            cpb.wait()
            wa = pltpu.async_copy(bufa, ga_hbm.at[pl.ds(chunk * _CK, _CK)], semo)
            wb = pltpu.async_copy(bufb, gb_hbm.at[pl.ds(chunk * _CK, _CK)], semo)
            wa.wait()
            wb.wait()

        return carry

    lax.fori_loop(0, _NJ, chunk_step, 0, unroll=False)


@functools.partial(jax.jit, donate_argnums=())
def _sc_gather(hA, hB, row2d, col2d):
    mesh = plsc.VectorSubcoreMesh(core_axis_name="c", subcore_axis_name="s")
    f = pl.kernel(
        _gather_body,
        mesh=mesh,
        out_type=[
            jax.ShapeDtypeStruct((E, H // 2), jnp.int32),
            jax.ShapeDtypeStruct((E, H // 2), jnp.int32),
        ],
        scratch_types=[
            pltpu.VMEM((1, _CK), jnp.int32),
            pltpu.VMEM((1, _CK), jnp.int32),
            pltpu.VMEM((_CK, H // 2), jnp.int32),
            pltpu.VMEM((_CK, H // 2), jnp.int32),
            pltpu.SemaphoreType.DMA,
            pltpu.SemaphoreType.DMA,
            pltpu.SemaphoreType.DMA,
        ],
        compiler_params=pltpu.CompilerParams(use_tc_tiling_on_sc=False),
    )
    return f(hA, hB, row2d, col2d)


def _mlp_body(ga_ref, gb_ref, cd_ref, w1_ref, w2_ref, b2_ref, wc1_ref, bc1_ref,
              wc2t_ref, out_ref):
    cd = cd_ref[...]
    rad = jnp.sum(cd * cd, axis=1, keepdims=True)
    w1r = w1_ref[2 * D:2 * D + 1, :]
    g = ga_ref[...].astype(jnp.float32) + gb_ref[...].astype(jnp.float32)
    x1 = _silu(g + rad * w1r)
    x2 = _silu(jnp.dot(x1.astype(jnp.bfloat16), w2_ref[...],
                       preferred_element_type=jnp.float32) + b2_ref[...])
    x3 = _silu(jnp.dot(x2.astype(jnp.bfloat16), wc1_ref[...],
                       preferred_element_type=jnp.float32) + bc1_ref[...])
    s = jnp.sum(x3 * wc2t_ref[...], axis=1, keepdims=True)
    t = jnp.clip(cd * s, -100.0, 100.0)
    ones = jnp.ones((t.shape[0], 1), jnp.float32)
    t4 = jnp.concatenate([t, ones], axis=1)
    out_ref[...] = t4.T


def _edge_mlp(gA, gB, coord_diff, W1, W2, b2, Wc1, bc1, Wc2):
    BE = 2560
    wc2t = Wc2.reshape(1, H)
    return pl.pallas_call(
        _mlp_body,
        grid=(E // BE,),
        in_specs=[
            pl.BlockSpec((BE, H), lambda i: (i, 0)),
            pl.BlockSpec((BE, H), lambda i: (i, 0)),
            pl.BlockSpec((BE, 3), lambda i: (i, 0)),
            pl.BlockSpec((2 * D + 1, H), lambda i: (0, 0)),
            pl.BlockSpec((H, H), lambda i: (0, 0)),
            pl.BlockSpec((1, H), lambda i: (0, 0)),
            pl.BlockSpec((H, H), lambda i: (0, 0)),
            pl.BlockSpec((1, H), lambda i: (0, 0)),
            pl.BlockSpec((1, H), lambda i: (0, 0)),
        ],
        out_specs=pl.BlockSpec((4, BE), lambda i: (0, i)),
        out_shape=jax.ShapeDtypeStruct((4, E), jnp.float32),
    )(gA, gB, coord_diff, W1, W2.astype(jnp.bfloat16), b2.reshape(1, H),
      Wc1.astype(jnp.bfloat16), bc1.reshape(1, H), wc2t)


def _scatter_body(trans4t_hbm, row2d_hbm, zeros_hbm, out_hbm,
                  idxv, idx4, tbuf, vbuf, acc):
    cid = lax.axis_index("c")
    sid = lax.axis_index("s")
    wid = sid * _NC + cid

    @pl.when(sid < 10)
    def _():
        pltpu.sync_copy(zeros_hbm.at[pl.ds(sid * 4000, 4000)], vbuf)
        pltpu.sync_copy(vbuf, acc.at[pl.ds(sid * 4000, 4000)])

    plsc.subcore_barrier()

    def chunk_step(j, carry):
        chunk = wid + _NW * j

        @pl.when(chunk < _NCHUNK)
        def _():
            pltpu.sync_copy(row2d_hbm.at[pl.ds(chunk, 1)], idxv)
            pltpu.sync_copy(trans4t_hbm.at[:, pl.ds(chunk * _CK, _CK)], tbuf)
            for gj in range(_CK // 16):
                sl = pl.ds(gj * 16, 16)
                r4 = idxv[0, sl] * 4
                for k in range(4):
                    idx4[k, sl] = r4 + k
            for k in range(4):
                pltpu.sync_copy(tbuf.at[k], acc.at[idx4.at[k]], add=True)

        return carry

    lax.fori_loop(0, _NJ, chunk_step, 0, unroll=False)
    plsc.subcore_barrier()

    @pl.when(sid < 10)
    def _():
        pltpu.sync_copy(acc.at[pl.ds(sid * 4000, 4000)], vbuf)
        pltpu.sync_copy(vbuf, out_hbm.at[pl.ds(cid * 4 * N + sid * 4000, 4000)])


@jax.jit
def _sc_scatter(trans4t, row2d, zeros_flat):
    mesh = plsc.VectorSubcoreMesh(core_axis_name="c", subcore_axis_name="s")
    f = pl.kernel(
        _scatter_body,
        mesh=mesh,
        out_type=jax.ShapeDtypeStruct((2 * 4 * N,), jnp.float32),
        scratch_types=[
            pltpu.VMEM((1, _CK), jnp.int32),
            pltpu.VMEM((4, _CK), jnp.int32),
            pltpu.VMEM((4, _CK), jnp.float32),
            pltpu.VMEM((4000,), jnp.float32),
            pltpu.VMEM_SHARED((4 * N,), jnp.float32),
        ],
    )
    return f(trans4t, row2d, zeros_flat)


def _combine_body(p_ref, out_ref):
    p = p_ref[0] + p_ref[1]
    cnt = jnp.maximum(p[:, 3:4], 1.0)
    out_ref[...] = p[:, 0:3] / cnt


def _combine(partials):
    return pl.pallas_call(
        _combine_body,
        grid=(1,),
        in_specs=[pl.BlockSpec((2, N, 4), lambda i: (0, 0, 0))],
        out_specs=pl.BlockSpec((N, 3), lambda i: (0, 0)),
        out_shape=jax.ShapeDtypeStruct((N, 3), jnp.float32),
    )(partials)


def kernel(h, coord_diff, edge_index, W1, b1, W2, b2, Wc1, bc1, Wc2):
    row = edge_index[0]
    col = edge_index[1]
    hA, hB = _proj(h, W1, b1.reshape(1, H))
    hA32 = lax.bitcast_convert_type(hA.reshape(N, H // 2, 2), jnp.int32)
    hB32 = lax.bitcast_convert_type(hB.reshape(N, H // 2, 2), jnp.int32)
    row2d = row.reshape(_NCHUNK, _CK)
    col2d = col.reshape(_NCHUNK, _CK)
    gA32, gB32 = _sc_gather(hA32, hB32, row2d, col2d)
    gA = lax.bitcast_convert_type(gA32, jnp.bfloat16).reshape(E, H)
    gB = lax.bitcast_convert_type(gB32, jnp.bfloat16).reshape(E, H)
    trans4t = _edge_mlp(gA, gB, coord_diff, W1, W2, b2, Wc1, bc1, Wc2)
    zeros_flat = jnp.zeros((4 * N,), jnp.float32)
    partials = _sc_scatter(trans4t, row2d, zeros_flat).reshape(2, N, 4)
    return _combine(partials)


# speedup vs baseline: 3.7740x; 3.7740x over previous
import functools

import jax
import jax.numpy as jnp
from jax import lax
from jax.experimental import pallas as pl
from jax.experimental.pallas import tpu as pltpu
from jax.experimental.pallas import tpu_sc as plsc

N = 10000
E = 320000
D = 128
H = 128

_NC = 2
_NS = 16
_NW = _NC * _NS
_CK = 128
_NCHUNK = E // _CK
_NJ = (_NCHUNK + _NW - 1) // _NW
_NR = (_NJ + 1) // 2


def _silu(x):
    return x * jax.nn.sigmoid(x)


def _proj_body(h_ref, w1_ref, b1_ref, ha_ref, hb_ref):
    h = h_ref[...]
    w1a = w1_ref[0:D, :]
    w1b = w1_ref[D:2 * D, :]
    ha_ref[...] = jnp.dot(h, w1a, preferred_element_type=jnp.float32) + b1_ref[...]
    hb_ref[...] = jnp.dot(h, w1b, preferred_element_type=jnp.float32)


def _proj(h, W1, b1):
    BN = 2000
    return pl.pallas_call(
        _proj_body,
        grid=(N // BN,),
        in_specs=[
            pl.BlockSpec((BN, D), lambda i: (i, 0)),
            pl.BlockSpec((2 * D + 1, H), lambda i: (0, 0)),
            pl.BlockSpec((1, H), lambda i: (0, 0)),
        ],
        out_specs=[
            pl.BlockSpec((BN, H), lambda i: (i, 0)),
            pl.BlockSpec((BN, H), lambda i: (i, 0)),
        ],
        out_shape=[
            jax.ShapeDtypeStruct((N, H), jnp.float32),
            jax.ShapeDtypeStruct((N, H), jnp.float32),
        ],
    )(h, W1, b1)


def _gather_body(ha_hbm, hb_hbm, row2d_hbm, col2d_hbm, g_hbm,
                 rowv0, colv0, rowv1, colv1,
                 bufa0, bufb0, bufo0, bufa1, bufb1, bufo1,
                 sga0, sgb0, swo0, sga1, sgb1, swo1):
    wid = lax.axis_index("s") * _NC + lax.axis_index("c")

    slots = (
        (rowv0, colv0, bufa0, bufb0, bufo0, sga0, sgb0, swo0),
        (rowv1, colv1, bufa1, bufb1, bufo1, sga1, sgb1, swo1),
    )

    def stage_idx(j, slot):
        rowv, colv = slots[slot][0], slots[slot][1]
        chunk = wid + _NW * j

        @pl.when(chunk < _NCHUNK)
        def _():
            pltpu.sync_copy(row2d_hbm.at[pl.ds(chunk, 1)], rowv)
            pltpu.sync_copy(col2d_hbm.at[pl.ds(chunk, 1)], colv)

    def issue(j, slot):
        rowv, colv, bufa, bufb, _, sga, sgb, _ = slots[slot]
        chunk = wid + _NW * j

        @pl.when(chunk < _NCHUNK)
        def _():
            pltpu.make_async_copy(ha_hbm.at[rowv.at[0]], bufa, sga).start()
            pltpu.make_async_copy(hb_hbm.at[colv.at[0]], bufb, sgb).start()

    def process(j, slot, have_prev_wo):
        rowv, colv, bufa, bufb, bufo, sga, sgb, swo = slots[slot]
        chunk = wid + _NW * j

        if have_prev_wo:
            prev = wid + _NW * (j - 2)

            @pl.when(jnp.logical_and(prev >= 0, prev < _NCHUNK))
            def _():
                pltpu.make_async_copy(bufo, g_hbm.at[pl.ds(0, _CK)], swo).wait()

        @pl.when(chunk < _NCHUNK)
        def _():
            pltpu.make_async_copy(ha_hbm.at[rowv.at[0]], bufa, sga).wait()
            pltpu.make_async_copy(hb_hbm.at[colv.at[0]], bufb, sgb).wait()

            def add_row(r, carry2):
                for cc in range(H // 16):
                    sl = pl.ds(cc * 16, 16)
                    bufo[r, sl] = bufa[r, sl] + bufb[r, sl]
                return carry2

            lax.fori_loop(0, _CK, add_row, 0, unroll=False)
            pltpu.make_async_copy(bufo, g_hbm.at[pl.ds(chunk * _CK, _CK)],
                                  swo).start()

    stage_idx(0, 0)
    issue(0, 0)
    stage_idx(1, 1)

    def round_step(r, carry):
        j0 = 2 * r
        j1 = 2 * r + 1
        issue(j1, 1)
        process(j0, 0, True)
        stage_idx(j0 + 2, 0)
        issue(j0 + 2, 0)
        process(j1, 1, True)
        stage_idx(j1 + 2, 1)
        return carry

    lax.fori_loop(0, _NR, round_step, 0, unroll=False)

    last0 = 2 * (_NR - 1)
    last1 = last0 + 1

    @pl.when(wid + _NW * last0 < _NCHUNK)
    def _():
        pltpu.make_async_copy(bufo0, g_hbm.at[pl.ds(0, _CK)], swo0).wait()

    @pl.when(wid + _NW * last1 < _NCHUNK)
    def _():
        pltpu.make_async_copy(bufo1, g_hbm.at[pl.ds(0, _CK)], swo1).wait()


@jax.jit
def _sc_gather(hA, hB, row2d, col2d):
    mesh = plsc.VectorSubcoreMesh(core_axis_name="c", subcore_axis_name="s")
    f = pl.kernel(
        _gather_body,
        mesh=mesh,
        out_type=jax.ShapeDtypeStruct((E, H), jnp.float32),
        scratch_types=[
            pltpu.VMEM((1, _CK), jnp.int32),
            pltpu.VMEM((1, _CK), jnp.int32),
            pltpu.VMEM((1, _CK), jnp.int32),
            pltpu.VMEM((1, _CK), jnp.int32),
            pltpu.VMEM((_CK, H), jnp.float32),
            pltpu.VMEM((_CK, H), jnp.float32),
            pltpu.VMEM((_CK, H), jnp.float32),
            pltpu.VMEM((_CK, H), jnp.float32),
            pltpu.VMEM((_CK, H), jnp.float32),
            pltpu.VMEM((_CK, H), jnp.float32),
            pltpu.SemaphoreType.DMA,
            pltpu.SemaphoreType.DMA,
            pltpu.SemaphoreType.DMA,
            pltpu.SemaphoreType.DMA,
            pltpu.SemaphoreType.DMA,
            pltpu.SemaphoreType.DMA,
        ],
    )
    return f(hA, hB, row2d, col2d)


def _mlp_body(g_ref, cd_ref, w1_ref, w2_ref, b2_ref, wc1_ref, bc1_ref,
              wc2t_ref, out_ref):
    cd = cd_ref[...]
    rad = jnp.sum(cd * cd, axis=1, keepdims=True)
    w1r = w1_ref[2 * D:2 * D + 1, :]
    x1 = _silu(g_ref[...] + rad * w1r)
    x2 = _silu(jnp.dot(x1.astype(jnp.bfloat16), w2_ref[...],
                       preferred_element_type=jnp.float32) + b2_ref[...])
    x3 = _silu(jnp.dot(x2.astype(jnp.bfloat16), wc1_ref[...],
                       preferred_element_type=jnp.float32) + bc1_ref[...])
    s = jnp.sum(x3 * wc2t_ref[...], axis=1, keepdims=True)
    t = jnp.clip(cd * s, -100.0, 100.0)
    ones = jnp.ones((t.shape[0], 1), jnp.float32)
    t4 = jnp.concatenate([t, ones], axis=1)
    out_ref[...] = t4.T


def _edge_mlp(g, coord_diff, W1, W2, b2, Wc1, bc1, Wc2):
    BE = 2560
    wc2t = Wc2.reshape(1, H)
    return pl.pallas_call(
        _mlp_body,
        grid=(E // BE,),
        in_specs=[
            pl.BlockSpec((BE, H), lambda i: (i, 0)),
            pl.BlockSpec((BE, 3), lambda i: (i, 0)),
            pl.BlockSpec((2 * D + 1, H), lambda i: (0, 0)),
            pl.BlockSpec((H, H), lambda i: (0, 0)),
            pl.BlockSpec((1, H), lambda i: (0, 0)),
            pl.BlockSpec((H, H), lambda i: (0, 0)),
            pl.BlockSpec((1, H), lambda i: (0, 0)),
            pl.BlockSpec((1, H), lambda i: (0, 0)),
        ],
        out_specs=pl.BlockSpec((4, BE), lambda i: (0, i)),
        out_shape=jax.ShapeDtypeStruct((4, E), jnp.float32),
    )(g, coord_diff, W1, W2.astype(jnp.bfloat16), b2.reshape(1, H),
      Wc1.astype(jnp.bfloat16), bc1.reshape(1, H), wc2t)


def _scatter_body(trans4t_hbm, row2d_hbm, zeros_hbm, out_hbm,
                  idxv, idx4, tbuf, vbuf, acc):
    cid = lax.axis_index("c")
    sid = lax.axis_index("s")
    wid = sid * _NC + cid

    @pl.when(sid < 10)
    def _():
        pltpu.sync_copy(zeros_hbm.at[pl.ds(sid * 4000, 4000)], vbuf)
        pltpu.sync_copy(vbuf, acc.at[pl.ds(sid * 4000, 4000)])

    plsc.subcore_barrier()

    def chunk_step(j, carry):
        chunk = wid + _NW * j

        @pl.when(chunk < _NCHUNK)
        def _():
            pltpu.sync_copy(row2d_hbm.at[pl.ds(chunk, 1)], idxv)
            pltpu.sync_copy(trans4t_hbm.at[:, pl.ds(chunk * _CK, _CK)], tbuf)
            for gj in range(_CK // 16):
                sl = pl.ds(gj * 16, 16)
                r4 = idxv[0, sl] * 4
                for k in range(4):
                    idx4[k, sl] = r4 + k
            for k in range(4):
                pltpu.sync_copy(tbuf.at[k], acc.at[idx4.at[k]], add=True)

        return carry

    lax.fori_loop(0, _NJ, chunk_step, 0, unroll=False)
    plsc.subcore_barrier()

    @pl.when(sid < 10)
    def _():
        pltpu.sync_copy(acc.at[pl.ds(sid * 4000, 4000)], vbuf)
        pltpu.sync_copy(vbuf, out_hbm.at[pl.ds(cid * 4 * N + sid * 4000, 4000)])


@jax.jit
def _sc_scatter(trans4t, row2d, zeros_flat):
    mesh = plsc.VectorSubcoreMesh(core_axis_name="c", subcore_axis_name="s")
    f = pl.kernel(
        _scatter_body,
        mesh=mesh,
        out_type=jax.ShapeDtypeStruct((2 * 4 * N,), jnp.float32),
        scratch_types=[
            pltpu.VMEM((1, _CK), jnp.int32),
            pltpu.VMEM((4, _CK), jnp.int32),
            pltpu.VMEM((4, _CK), jnp.float32),
            pltpu.VMEM((4000,), jnp.float32),
            pltpu.VMEM_SHARED((4 * N,), jnp.float32),
        ],
    )
    return f(trans4t, row2d, zeros_flat)


def _combine_body(p_ref, out_ref):
    p = p_ref[0] + p_ref[1]
    cnt = jnp.maximum(p[:, 3:4], 1.0)
    out_ref[...] = p[:, 0:3] / cnt


def _combine(partials):
    return pl.pallas_call(
        _combine_body,
        grid=(1,),
        in_specs=[pl.BlockSpec((2, N, 4), lambda i: (0, 0, 0))],
        out_specs=pl.BlockSpec((N, 3), lambda i: (0, 0)),
        out_shape=jax.ShapeDtypeStruct((N, 3), jnp.float32),
    )(partials)


def kernel(h, coord_diff, edge_index, W1, b1, W2, b2, Wc1, bc1, Wc2):
    row = edge_index[0]
    col = edge_index[1]
    hA, hB = _proj(h, W1, b1.reshape(1, H))
    row2d = row.reshape(_NCHUNK, _CK)
    col2d = col.reshape(_NCHUNK, _CK)
    g = _sc_gather(hA, hB, row2d, col2d)
    trans4t = _edge_mlp(g, coord_diff, W1, W2, b2, Wc1, bc1, Wc2)
    zeros_flat = jnp.zeros((4 * N,), jnp.float32)
    partials = _sc_scatter(trans4t, row2d, zeros_flat).reshape(2, N, 4)
    return _combine(partials)


# speedup vs baseline: 4.3017x; 1.1398x over previous
import functools

import jax
import jax.numpy as jnp
from jax import lax
from jax.experimental import pallas as pl
from jax.experimental.pallas import tpu as pltpu
from jax.experimental.pallas import tpu_sc as plsc

N = 10000
E = 320000
D = 128
H = 128

_NC = 2
_NS = 16
_NW = _NC * _NS
_CK = 128
_NCHUNK = E // _CK
_NJ = (_NCHUNK + _NW - 1) // _NW
_NR = (_NJ + 1) // 2


def _silu(x):
    return x * jax.nn.sigmoid(x)


def _proj_body(h_ref, w1_ref, b1_ref, ha_ref, hb_ref):
    h = h_ref[...]
    w1a = w1_ref[0:D, :]
    w1b = w1_ref[D:2 * D, :]
    ha_ref[...] = jnp.dot(h, w1a, preferred_element_type=jnp.float32) + b1_ref[...]
    hb_ref[...] = jnp.dot(h, w1b, preferred_element_type=jnp.float32)


def _proj(h, W1, b1):
    BN = 2000
    return pl.pallas_call(
        _proj_body,
        grid=(N // BN,),
        in_specs=[
            pl.BlockSpec((BN, D), lambda i: (i, 0)),
            pl.BlockSpec((2 * D + 1, H), lambda i: (0, 0)),
            pl.BlockSpec((1, H), lambda i: (0, 0)),
        ],
        out_specs=[
            pl.BlockSpec((BN, H), lambda i: (i, 0)),
            pl.BlockSpec((BN, H), lambda i: (i, 0)),
        ],
        out_shape=[
            jax.ShapeDtypeStruct((N, H), jnp.float32),
            jax.ShapeDtypeStruct((N, H), jnp.float32),
        ],
    )(h, W1, b1)


def _gather_body(ha_hbm, hb_hbm, row2d_hbm, col2d_hbm, g_hbm,
                 rowv0, colv0, rowv1, colv1,
                 bufa0, bufb0, bufo0, bufa1, bufb1, bufo1,
                 sga0, sgb0, swo0, sga1, sgb1, swo1):
    wid = lax.axis_index("s") * _NC + lax.axis_index("c")

    slots = (
        (rowv0, colv0, bufa0, bufb0, bufo0, sga0, sgb0, swo0),
        (rowv1, colv1, bufa1, bufb1, bufo1, sga1, sgb1, swo1),
    )

    def stage_idx(j, slot):
        rowv, colv = slots[slot][0], slots[slot][1]
        chunk = wid + _NW * j

        @pl.when(chunk < _NCHUNK)
        def _():
            pltpu.sync_copy(row2d_hbm.at[pl.ds(chunk, 1)], rowv)
            pltpu.sync_copy(col2d_hbm.at[pl.ds(chunk, 1)], colv)

    def issue(j, slot):
        rowv, colv, bufa, bufb, _, sga, sgb, _ = slots[slot]
        chunk = wid + _NW * j

        @pl.when(chunk < _NCHUNK)
        def _():
            pltpu.make_async_copy(ha_hbm.at[rowv.at[0]], bufa, sga).start()
            pltpu.make_async_copy(hb_hbm.at[colv.at[0]], bufb, sgb).start()

    def process(j, slot, have_prev_wo):
        rowv, colv, bufa, bufb, bufo, sga, sgb, swo = slots[slot]
        chunk = wid + _NW * j

        if have_prev_wo:
            prev = wid + _NW * (j - 2)

            @pl.when(jnp.logical_and(prev >= 0, prev < _NCHUNK))
            def _():
                pltpu.make_async_copy(bufo, g_hbm.at[pl.ds(0, _CK)], swo).wait()

        @pl.when(chunk < _NCHUNK)
        def _():
            pltpu.make_async_copy(ha_hbm.at[rowv.at[0]], bufa, sga).wait()
            pltpu.make_async_copy(hb_hbm.at[colv.at[0]], bufb, sgb).wait()

            def add_row(r, carry2):
                for cc in range(H // 16):
                    sl = pl.ds(cc * 16, 16)
                    bufo[r, sl] = bufa[r, sl] + bufb[r, sl]
                return carry2

            lax.fori_loop(0, _CK, add_row, 0, unroll=False)
            pltpu.make_async_copy(bufo, g_hbm.at[pl.ds(chunk * _CK, _CK)],
                                  swo).start()

    stage_idx(0, 0)
    issue(0, 0)
    stage_idx(1, 1)

    def round_step(r, carry):
        j0 = 2 * r
        j1 = 2 * r + 1
        issue(j1, 1)
        process(j0, 0, True)
        stage_idx(j0 + 2, 0)
        issue(j0 + 2, 0)
        process(j1, 1, True)
        stage_idx(j1 + 2, 1)
        return carry

    lax.fori_loop(0, _NR, round_step, 0, unroll=False)

    last0 = 2 * (_NR - 1)
    last1 = last0 + 1

    @pl.when(wid + _NW * last0 < _NCHUNK)
    def _():
        pltpu.make_async_copy(bufo0, g_hbm.at[pl.ds(0, _CK)], swo0).wait()

    @pl.when(wid + _NW * last1 < _NCHUNK)
    def _():
        pltpu.make_async_copy(bufo1, g_hbm.at[pl.ds(0, _CK)], swo1).wait()


@jax.jit
def _sc_gather(hA, hB, row2d, col2d):
    mesh = plsc.VectorSubcoreMesh(core_axis_name="c", subcore_axis_name="s")
    f = pl.kernel(
        _gather_body,
        mesh=mesh,
        out_type=jax.ShapeDtypeStruct((E, H), jnp.float32),
        scratch_types=[
            pltpu.VMEM((1, _CK), jnp.int32),
            pltpu.VMEM((1, _CK), jnp.int32),
            pltpu.VMEM((1, _CK), jnp.int32),
            pltpu.VMEM((1, _CK), jnp.int32),
            pltpu.VMEM((_CK, H), jnp.float32),
            pltpu.VMEM((_CK, H), jnp.float32),
            pltpu.VMEM((_CK, H), jnp.float32),
            pltpu.VMEM((_CK, H), jnp.float32),
            pltpu.VMEM((_CK, H), jnp.float32),
            pltpu.VMEM((_CK, H), jnp.float32),
            pltpu.SemaphoreType.DMA,
            pltpu.SemaphoreType.DMA,
            pltpu.SemaphoreType.DMA,
            pltpu.SemaphoreType.DMA,
            pltpu.SemaphoreType.DMA,
            pltpu.SemaphoreType.DMA,
        ],
    )
    return f(hA, hB, row2d, col2d)


def _mlp_body(g_ref, cd_ref, w1_ref, w2_ref, b2_ref, wc1_ref, bc1_ref,
              wc2t_ref, out_ref):
    cd = cd_ref[...]
    rad = jnp.sum(cd * cd, axis=1, keepdims=True)
    w1r = w1_ref[2 * D:2 * D + 1, :]
    x1 = _silu(g_ref[...] + rad * w1r)
    x2 = _silu(jnp.dot(x1.astype(jnp.bfloat16), w2_ref[...],
                       preferred_element_type=jnp.float32) + b2_ref[...])
    x3 = _silu(jnp.dot(x2.astype(jnp.bfloat16), wc1_ref[...],
                       preferred_element_type=jnp.float32) + bc1_ref[...])
    s = jnp.sum(x3 * wc2t_ref[...], axis=1, keepdims=True)
    t = jnp.clip(cd * s, -100.0, 100.0)
    ones = jnp.ones((t.shape[0], 1), jnp.float32)
    t4 = jnp.concatenate([t, ones], axis=1)
    out_ref[...] = t4.T


def _edge_mlp(g, coord_diff, W1, W2, b2, Wc1, bc1, Wc2):
    BE = 2560
    wc2t = Wc2.reshape(1, H)
    return pl.pallas_call(
        _mlp_body,
        grid=(E // BE,),
        in_specs=[
            pl.BlockSpec((BE, H), lambda i: (i, 0)),
            pl.BlockSpec((BE, 3), lambda i: (i, 0)),
            pl.BlockSpec((2 * D + 1, H), lambda i: (0, 0)),
            pl.BlockSpec((H, H), lambda i: (0, 0)),
            pl.BlockSpec((1, H), lambda i: (0, 0)),
            pl.BlockSpec((H, H), lambda i: (0, 0)),
            pl.BlockSpec((1, H), lambda i: (0, 0)),
            pl.BlockSpec((1, H), lambda i: (0, 0)),
        ],
        out_specs=pl.BlockSpec((4, BE), lambda i: (0, i)),
        out_shape=jax.ShapeDtypeStruct((4, E), jnp.float32),
    )(g, coord_diff, W1, W2.astype(jnp.bfloat16), b2.reshape(1, H),
      Wc1.astype(jnp.bfloat16), bc1.reshape(1, H), wc2t)


_SB = 4
_NB = (_NJ + _SB - 1) // _SB


def _scatter_body(trans4t_hbm, row2d_hbm, zeros_hbm, out_hbm,
                  idxv, idx16, tbuf16, vbuf, acc, sst, ssc):
    cid = lax.axis_index("c")
    sid = lax.axis_index("s")
    wid = sid * _NC + cid

    @pl.when(sid < 10)
    def _():
        pltpu.sync_copy(zeros_hbm.at[pl.ds(sid * 4000, 4000)], vbuf)
        pltpu.sync_copy(vbuf, acc.at[pl.ds(sid * 4000, 4000)])

    plsc.subcore_barrier()

    def batch_step(b, carry):
        stage = []
        for u in range(_SB):
            chunk = wid + _NW * (b * _SB + u)
            chunkc = jnp.minimum(chunk, _NCHUNK - 1)
            stage.append(pltpu.async_copy(
                row2d_hbm.at[pl.ds(chunkc, 1)], idxv.at[pl.ds(u, 1)], sst))
            stage.append(pltpu.async_copy(
                trans4t_hbm.at[:, pl.ds(chunkc * _CK, _CK)],
                tbuf16.at[pl.ds(u * 4, 4)], sst))
        for hh in stage:
            hh.wait()
        for u in range(_SB):
            chunk = wid + _NW * (b * _SB + u)
            valid = chunk < _NCHUNK
            for gj in range(_CK // 16):
                sl = pl.ds(gj * 16, 16)
                r4 = jnp.where(valid, idxv[u, sl] * 4, 4 * N)
                for k in range(4):
                    idx16[u * 4 + k, sl] = r4 + k
        scs = []
        for m in range(4 * _SB):
            scs.append(pltpu.async_copy(
                tbuf16.at[m], acc.at[idx16.at[m]], ssc, add=True))
        for s in scs:
            s.wait()
        return carry

    lax.fori_loop(0, _NB, batch_step, 0, unroll=False)
    plsc.subcore_barrier()

    @pl.when(sid < 10)
    def _():
        pltpu.sync_copy(acc.at[pl.ds(sid * 4000, 4000)], vbuf)
        pltpu.sync_copy(vbuf, out_hbm.at[pl.ds(cid * 4 * N + sid * 4000, 4000)])


@jax.jit
def _sc_scatter(trans4t, row2d, zeros_flat):
    mesh = plsc.VectorSubcoreMesh(core_axis_name="c", subcore_axis_name="s")
    f = pl.kernel(
        _scatter_body,
        mesh=mesh,
        out_type=jax.ShapeDtypeStruct((2 * 4 * N,), jnp.float32),
        scratch_types=[
            pltpu.VMEM((_SB, _CK), jnp.int32),
            pltpu.VMEM((4 * _SB, _CK), jnp.int32),
            pltpu.VMEM((4 * _SB, _CK), jnp.float32),
            pltpu.VMEM((4000,), jnp.float32),
            pltpu.VMEM_SHARED((4 * N + 16,), jnp.float32),
            pltpu.SemaphoreType.DMA,
            pltpu.SemaphoreType.DMA,
        ],
    )
    return f(trans4t, row2d, zeros_flat)


def _combine_body(p_ref, out_ref):
    p = p_ref[0] + p_ref[1]
    cnt = jnp.maximum(p[:, 3:4], 1.0)
    out_ref[...] = p[:, 0:3] / cnt


def _combine(partials):
    return pl.pallas_call(
        _combine_body,
        grid=(1,),
        in_specs=[pl.BlockSpec((2, N, 4), lambda i: (0, 0, 0))],
        out_specs=pl.BlockSpec((N, 3), lambda i: (0, 0)),
        out_shape=jax.ShapeDtypeStruct((N, 3), jnp.float32),
    )(partials)


def kernel(h, coord_diff, edge_index, W1, b1, W2, b2, Wc1, bc1, Wc2):
    row = edge_index[0]
    col = edge_index[1]
    hA, hB = _proj(h, W1, b1.reshape(1, H))
    row2d = row.reshape(_NCHUNK, _CK)
    col2d = col.reshape(_NCHUNK, _CK)
    g = _sc_gather(hA, hB, row2d, col2d)
    trans4t = _edge_mlp(g, coord_diff, W1, W2, b2, Wc1, bc1, Wc2)
    zeros_flat = jnp.zeros((4 * N,), jnp.float32)
    partials = _sc_scatter(trans4t, row2d, zeros_flat).reshape(2, N, 4)
    return _combine(partials)


# speedup vs baseline: 4.6972x; 1.0919x over previous
import functools

import jax
import jax.numpy as jnp
from jax import lax
from jax.experimental import pallas as pl
from jax.experimental.pallas import tpu as pltpu
from jax.experimental.pallas import tpu_sc as plsc

N = 10000
E = 320000
D = 128
H = 128

_NC = 2
_NS = 16
_NW = _NC * _NS
_CK = 128
_NCHUNK = E // _CK
_NJ = (_NCHUNK + _NW - 1) // _NW
_NR = (_NJ + 1) // 2


def _silu(x):
    return x * jax.nn.sigmoid(x)


def _proj_body(h_ref, w1_ref, b1_ref, ha_ref, hb_ref):
    h = h_ref[...]
    w1a = w1_ref[0:D, :]
    w1b = w1_ref[D:2 * D, :]
    ha_ref[...] = jnp.dot(h, w1a, preferred_element_type=jnp.float32) + b1_ref[...]
    hb_ref[...] = jnp.dot(h, w1b, preferred_element_type=jnp.float32)


def _proj(h, W1, b1):
    BN = 2000
    return pl.pallas_call(
        _proj_body,
        grid=(N // BN,),
        in_specs=[
            pl.BlockSpec((BN, D), lambda i: (i, 0)),
            pl.BlockSpec((2 * D + 1, H), lambda i: (0, 0)),
            pl.BlockSpec((1, H), lambda i: (0, 0)),
        ],
        out_specs=[
            pl.BlockSpec((BN, H), lambda i: (i, 0)),
            pl.BlockSpec((BN, H), lambda i: (i, 0)),
        ],
        out_shape=[
            jax.ShapeDtypeStruct((N, H), jnp.float32),
            jax.ShapeDtypeStruct((N, H), jnp.float32),
        ],
    )(h, W1, b1)


def _gather_body(nchunk, ha_hbm, hb_hbm, row2d_hbm, col2d_hbm, g_hbm,
                 rowv0, colv0, rowv1, colv1,
                 bufa0, bufb0, bufo0, bufa1, bufb1, bufo1,
                 sga0, sgb0, swo0, sga1, sgb1, swo1):
    nj = (nchunk + _NW - 1) // _NW
    nr = (nj + 1) // 2
    wid = lax.axis_index("s") * _NC + lax.axis_index("c")

    slots = (
        (rowv0, colv0, bufa0, bufb0, bufo0, sga0, sgb0, swo0),
        (rowv1, colv1, bufa1, bufb1, bufo1, sga1, sgb1, swo1),
    )

    def stage_idx(j, slot):
        rowv, colv = slots[slot][0], slots[slot][1]
        chunk = wid + _NW * j

        @pl.when(chunk < nchunk)
        def _():
            pltpu.sync_copy(row2d_hbm.at[pl.ds(chunk, 1)], rowv)
            pltpu.sync_copy(col2d_hbm.at[pl.ds(chunk, 1)], colv)

    def issue(j, slot):
        rowv, colv, bufa, bufb, _, sga, sgb, _ = slots[slot]
        chunk = wid + _NW * j

        @pl.when(chunk < nchunk)
        def _():
            pltpu.make_async_copy(ha_hbm.at[rowv.at[0]], bufa, sga).start()
            pltpu.make_async_copy(hb_hbm.at[colv.at[0]], bufb, sgb).start()

    def process(j, slot, have_prev_wo):
        rowv, colv, bufa, bufb, bufo, sga, sgb, swo = slots[slot]
        chunk = wid + _NW * j

        if have_prev_wo:
            prev = wid + _NW * (j - 2)

            @pl.when(jnp.logical_and(prev >= 0, prev < nchunk))
            def _():
                pltpu.make_async_copy(bufo, g_hbm.at[pl.ds(0, _CK)], swo).wait()

        @pl.when(chunk < nchunk)
        def _():
            pltpu.make_async_copy(ha_hbm.at[rowv.at[0]], bufa, sga).wait()
            pltpu.make_async_copy(hb_hbm.at[colv.at[0]], bufb, sgb).wait()

            def add_row(r, carry2):
                for cc in range(H // 16):
                    sl = pl.ds(cc * 16, 16)
                    bufo[r, sl] = bufa[r, sl] + bufb[r, sl]
                return carry2

            lax.fori_loop(0, _CK, add_row, 0, unroll=False)
            pltpu.make_async_copy(bufo, g_hbm.at[pl.ds(chunk * _CK, _CK)],
                                  swo).start()

    stage_idx(0, 0)
    issue(0, 0)
    stage_idx(1, 1)

    def round_step(r, carry):
        j0 = 2 * r
        j1 = 2 * r + 1
        issue(j1, 1)
        process(j0, 0, True)
        stage_idx(j0 + 2, 0)
        issue(j0 + 2, 0)
        process(j1, 1, True)
        stage_idx(j1 + 2, 1)
        return carry

    lax.fori_loop(0, nr, round_step, 0, unroll=False)

    last0 = 2 * (nr - 1)
    last1 = last0 + 1

    @pl.when(wid + _NW * last0 < nchunk)
    def _():
        pltpu.make_async_copy(bufo0, g_hbm.at[pl.ds(0, _CK)], swo0).wait()

    @pl.when(wid + _NW * last1 < nchunk)
    def _():
        pltpu.make_async_copy(bufo1, g_hbm.at[pl.ds(0, _CK)], swo1).wait()


def _sc_gather(hA, hB, row2d, col2d):
    nchunk = row2d.shape[0]
    mesh = plsc.VectorSubcoreMesh(core_axis_name="c", subcore_axis_name="s")
    f = pl.kernel(
        functools.partial(_gather_body, nchunk),
        mesh=mesh,
        out_type=jax.ShapeDtypeStruct((nchunk * _CK, H), jnp.float32),
        scratch_types=[
            pltpu.VMEM((1, _CK), jnp.int32),
            pltpu.VMEM((1, _CK), jnp.int32),
            pltpu.VMEM((1, _CK), jnp.int32),
            pltpu.VMEM((1, _CK), jnp.int32),
            pltpu.VMEM((_CK, H), jnp.float32),
            pltpu.VMEM((_CK, H), jnp.float32),
            pltpu.VMEM((_CK, H), jnp.float32),
            pltpu.VMEM((_CK, H), jnp.float32),
            pltpu.VMEM((_CK, H), jnp.float32),
            pltpu.VMEM((_CK, H), jnp.float32),
            pltpu.SemaphoreType.DMA,
            pltpu.SemaphoreType.DMA,
            pltpu.SemaphoreType.DMA,
            pltpu.SemaphoreType.DMA,
            pltpu.SemaphoreType.DMA,
            pltpu.SemaphoreType.DMA,
        ],
    )
    return f(hA, hB, row2d, col2d)


def _mlp_body(g_ref, cd_ref, w1_ref, w2_ref, b2_ref, wc1_ref, bc1_ref,
              wc2t_ref, out_ref):
    cd = cd_ref[...]
    rad = jnp.sum(cd * cd, axis=1, keepdims=True)
    w1r = w1_ref[2 * D:2 * D + 1, :]
    x1 = _silu(g_ref[...] + rad * w1r)
    x2 = _silu(jnp.dot(x1.astype(jnp.bfloat16), w2_ref[...],
                       preferred_element_type=jnp.float32) + b2_ref[...])
    x3 = _silu(jnp.dot(x2.astype(jnp.bfloat16), wc1_ref[...],
                       preferred_element_type=jnp.float32) + bc1_ref[...])
    s = jnp.sum(x3 * wc2t_ref[...], axis=1, keepdims=True)
    t = jnp.clip(cd * s, -100.0, 100.0)
    ones = jnp.ones((t.shape[0], 1), jnp.float32)
    t4 = jnp.concatenate([t, ones], axis=1)
    out_ref[...] = t4.T


def _edge_mlp(g, coord_diff, W1, W2, b2, Wc1, bc1, Wc2):
    BE = 3200
    ne = g.shape[0]
    wc2t = Wc2.reshape(1, H)
    return pl.pallas_call(
        _mlp_body,
        grid=(ne // BE,),
        in_specs=[
            pl.BlockSpec((BE, H), lambda i: (i, 0)),
            pl.BlockSpec((BE, 3), lambda i: (i, 0)),
            pl.BlockSpec((2 * D + 1, H), lambda i: (0, 0)),
            pl.BlockSpec((H, H), lambda i: (0, 0)),
            pl.BlockSpec((1, H), lambda i: (0, 0)),
            pl.BlockSpec((H, H), lambda i: (0, 0)),
            pl.BlockSpec((1, H), lambda i: (0, 0)),
            pl.BlockSpec((1, H), lambda i: (0, 0)),
        ],
        out_specs=pl.BlockSpec((4, BE), lambda i: (0, i)),
        out_shape=jax.ShapeDtypeStruct((4, ne), jnp.float32),
    )(g, coord_diff, W1, W2.astype(jnp.bfloat16), b2.reshape(1, H),
      Wc1.astype(jnp.bfloat16), bc1.reshape(1, H), wc2t)


_SB = 4
_NB = (_NJ + _SB - 1) // _SB


def _scatter_body(nchunk, trans4t_hbm, row2d_hbm, zeros_hbm, out_hbm,
                  idxv, idx16, tbuf16, vbuf, acc, sst, ssc):
    cid = lax.axis_index("c")
    sid = lax.axis_index("s")
    wid = sid * _NC + cid
    nj = (nchunk + _NW - 1) // _NW
    nb = (nj + _SB - 1) // _SB

    @pl.when(sid < 10)
    def _():
        pltpu.sync_copy(zeros_hbm.at[pl.ds(sid * 4000, 4000)], vbuf)
        pltpu.sync_copy(vbuf, acc.at[pl.ds(sid * 4000, 4000)])

    plsc.subcore_barrier()

    def batch_step(b, carry):
        stage = []
        for u in range(_SB):
            chunk = wid + _NW * (b * _SB + u)
            chunkc = jnp.minimum(chunk, nchunk - 1)
            stage.append(pltpu.async_copy(
                row2d_hbm.at[pl.ds(chunkc, 1)], idxv.at[pl.ds(u, 1)], sst))
            stage.append(pltpu.async_copy(
                trans4t_hbm.at[:, pl.ds(chunkc * _CK, _CK)],
                tbuf16.at[pl.ds(u * 4, 4)], sst))
        for hh in stage:
            hh.wait()
        for u in range(_SB):
            chunk = wid + _NW * (b * _SB + u)
            valid = chunk < nchunk
            for gj in range(_CK // 16):
                sl = pl.ds(gj * 16, 16)
                r4 = jnp.where(valid, idxv[u, sl] * 4, 4 * N)
                for k in range(4):
                    idx16[u * 4 + k, sl] = r4 + k
        scs = []
        for m in range(4 * _SB):
            scs.append(pltpu.async_copy(
                tbuf16.at[m], acc.at[idx16.at[m]], ssc, add=True))
        for s in scs:
            s.wait()
        return carry

    lax.fori_loop(0, nb, batch_step, 0, unroll=False)
    plsc.subcore_barrier()

    @pl.when(sid < 10)
    def _():
        pltpu.sync_copy(acc.at[pl.ds(sid * 4000, 4000)], vbuf)
        pltpu.sync_copy(vbuf, out_hbm.at[pl.ds(cid * 4 * N + sid * 4000, 4000)])


def _sc_scatter(trans4t, row2d, zeros_flat):
    nchunk = row2d.shape[0]
    mesh = plsc.VectorSubcoreMesh(core_axis_name="c", subcore_axis_name="s")
    f = pl.kernel(
        functools.partial(_scatter_body, nchunk),
        mesh=mesh,
        out_type=jax.ShapeDtypeStruct((2 * 4 * N,), jnp.float32),
        scratch_types=[
            pltpu.VMEM((_SB, _CK), jnp.int32),
            pltpu.VMEM((4 * _SB, _CK), jnp.int32),
            pltpu.VMEM((4 * _SB, _CK), jnp.float32),
            pltpu.VMEM((4000,), jnp.float32),
            pltpu.VMEM_SHARED((4 * N + 16,), jnp.float32),
            pltpu.SemaphoreType.DMA,
            pltpu.SemaphoreType.DMA,
        ],
    )
    return f(trans4t, row2d, zeros_flat)


def _combine_body(p_ref, out_ref):
    p = jnp.sum(p_ref[...], axis=0)
    cnt = jnp.maximum(p[:, 3:4], 1.0)
    out_ref[...] = p[:, 0:3] / cnt


def _combine(partials):
    npart = partials.shape[0]
    return pl.pallas_call(
        _combine_body,
        grid=(1,),
        in_specs=[pl.BlockSpec((npart, N, 4), lambda i: (0, 0, 0))],
        out_specs=pl.BlockSpec((N, 3), lambda i: (0, 0)),
        out_shape=jax.ShapeDtypeStruct((N, 3), jnp.float32),
    )(partials)


def kernel(h, coord_diff, edge_index, W1, b1, W2, b2, Wc1, bc1, Wc2):
    row = edge_index[0]
    col = edge_index[1]
    hA, hB = _proj(h, W1, b1.reshape(1, H))
    row2d = row.reshape(_NCHUNK, _CK)
    col2d = col.reshape(_NCHUNK, _CK)
    zeros_flat = jnp.zeros((4 * N,), jnp.float32)
    hc = _NCHUNK // 2
    eh = hc * _CK
    parts = []
    for lo in (0, hc):
        r2 = row2d[lo:lo + hc]
        c2 = col2d[lo:lo + hc]
        cd = coord_diff[lo * _CK:lo * _CK + eh]
        g = _sc_gather(hA, hB, r2, c2)
        t4t = _edge_mlp(g, cd, W1, W2, b2, Wc1, bc1, Wc2)
        parts.append(_sc_scatter(t4t, r2, zeros_flat))
    partials = jnp.concatenate(parts).reshape(2 * 2, N, 4)
    return _combine(partials)
